# baseline (speedup 1.0000x reference)
import functools

import jax
import jax.numpy as jnp
from jax import lax
from jax.experimental import pallas as pl
from jax.experimental.pallas import tpu as pltpu
from jax.experimental.pallas import tpu_sc as plsc

_D = 64
_S = 20
_NC = 2
_NS = 16
_NW = _NC * _NS

_K = 32
_ROWS = _K * _S
_IDXW = 128
_NG = _ROWS // _IDXW


def _sc_pool(story1d, c_tab, segs):
    segs_per_w = segs // _NW
    chunks = segs_per_w // _K

    mesh = plsc.VectorSubcoreMesh(core_axis_name="c", subcore_axis_name="s")

    @functools.partial(
        pl.kernel,
        mesh=mesh,
        compiler_params=pltpu.CompilerParams(
            use_tc_tiling_on_sc=False, needs_layout_passes=False
        ),
        out_type=jax.ShapeDtypeStruct((segs, _D), jnp.bfloat16),
        scratch_types=[
            pltpu.VMEM((_ROWS,), jnp.int32),
            pltpu.VMEM((_ROWS,), jnp.int32),
            pltpu.VMEM((_ROWS, _D), jnp.bfloat16),
            pltpu.VMEM((_ROWS, _D), jnp.bfloat16),
            pltpu.VMEM((_K, _D), jnp.bfloat16),
            pltpu.VMEM((_K, _D), jnp.bfloat16),
            pltpu.SemaphoreType.DMA,
            pltpu.SemaphoreType.DMA,
            pltpu.SemaphoreType.DMA,
            pltpu.SemaphoreType.DMA,
            pltpu.SemaphoreType.DMA,
            pltpu.SemaphoreType.DMA,
        ],
    )
    def k(story_hbm, c_hbm, out_hbm,
          idx0, idx1, rows0, rows1, outv0, outv1,
          sem_i0, sem_i1, sem_r0, sem_r1, sem_o0, sem_o1):
        idx_b = [idx0, idx1]
        rows_b = [rows0, rows1]
        out_b = [outv0, outv1]
        sem_i = [sem_i0, sem_i1]
        sem_r = [sem_r0, sem_r1]
        sem_o = [sem_o0, sem_o1]

        wid = lax.axis_index("s") * _NC + lax.axis_index("c")
        seg0 = wid * segs_per_w

        def stage_idx(ci, b):
            pltpu.async_copy(
                story_hbm.at[pl.ds((seg0 + ci * _K) * _S, _ROWS)],
                idx_b[b], sem_i[b],
            )

        def fire_gathers(b):
            pltpu.make_async_copy(
                story_hbm.at[pl.ds(0, _ROWS)], idx_b[b], sem_i[b]
            ).wait()
            for g in range(_NG):
                pltpu.async_copy(
                    c_hbm.at[idx_b[b].at[pl.ds(g * _IDXW, _IDXW)]],
                    rows_b[b].at[pl.ds(g * _IDXW, _IDXW)],
                    sem_r[b],
                )

        def drain_gathers(b):
            for g in range(_NG):
                pltpu.make_async_copy(
                    c_hbm.at[idx_b[b].at[pl.ds(g * _IDXW, _IDXW)]],
                    rows_b[b].at[pl.ds(g * _IDXW, _IDXW)],
                    sem_r[b],
                ).wait()

        def pool(b, ci):
            rows_v = rows_b[b]
            out_v = out_b[b]

            def seg_body(j, _):
                base = j * _S
                for l in range(_D // 32):
                    sl = pl.ds(l * 32, 32)
                    a0, a1 = plsc.unpack(
                        rows_v[base, sl], format=plsc.PackFormat.INTERLEAVED
                    )
                    for s in range(1, _S):
                        r0, r1 = plsc.unpack(
                            rows_v[base + s, sl],
                            format=plsc.PackFormat.INTERLEAVED,
                        )
                        a0 = a0 + r0
                        a1 = a1 + r1
                    out_v[j, sl] = plsc.pack(
                        a0, a1, format=plsc.PackFormat.INTERLEAVED
                    )
                return 0

            lax.fori_loop(0, _K, seg_body, 0, unroll=False)
            pltpu.async_copy(
                out_v,
                out_hbm.at[pl.ds(seg0 + ci * _K, _K)],
                sem_o[b],
            )

        def wait_out(b, ci):
            pltpu.make_async_copy(
                out_b[b],
                out_hbm.at[pl.ds(seg0 + ci * _K, _K)],
                sem_o[b],
            ).wait()

        stage_idx(0, 0)
        stage_idx(1, 1)
        fire_gathers(0)

        def chunk2_body(h, _):
            ci = h * 2
            for b in range(2):
                c = ci + b
                drain_gathers(b)

                @pl.when(c + 2 < chunks)
                def _():
                    stage_idx(c + 2, b)

                @pl.when(c + 1 < chunks)
                def _():
                    fire_gathers(1 - b)

                @pl.when(c >= 2)
                def _():
                    wait_out(b, c - 2)

                pool(b, c)
            return 0

        lax.fori_loop(0, chunks // 2, chunk2_body, 0, unroll=False)
        wait_out(0, chunks - 2)
        wait_out(1, chunks - 1)

    return k(story1d, c_tab)


def _tc_hop(ea, ec, u):
    B, M, D = ea.shape
    BB = 128

    def body(ea_ref, ec_ref, h_ref, o_ref):
        u = h_ref[...]
        eaf = ea_ref[...].astype(jnp.float32)
        logit = jnp.sum(eaf * u[:, None, :], axis=2)
        mx = jnp.max(logit, axis=1, keepdims=True)
        w = jnp.exp(logit - mx)
        den = jnp.sum(w, axis=1)
        ecf = ec_ref[...].astype(jnp.float32)
        num = jnp.sum(ecf * w[:, :, None], axis=1)
        o_ref[...] = u + num / den[:, None]

    return pl.pallas_call(
        body,
        grid=(B // BB,),
        in_specs=[
            pl.BlockSpec((BB, M, D), lambda i: (i, 0, 0)),
            pl.BlockSpec((BB, M, D), lambda i: (i, 0, 0)),
            pl.BlockSpec((BB, D), lambda i: (i, 0)),
        ],
        out_specs=pl.BlockSpec((BB, D), lambda i: (i, 0)),
        out_shape=jax.ShapeDtypeStruct((B, D), jnp.float32),
    )(ea, ec, u)


def kernel(story, hidden, C):
    B, M, S = story.shape
    T, vocab, D = C.shape
    story1d = story.reshape(-1).astype(jnp.int32)
    Cb, story1d = lax.optimization_barrier(
        (C.astype(jnp.bfloat16), story1d)
    )
    e = [
        _sc_pool(story1d, Cb[t], B * M).reshape(B, M, D)
        for t in range(T)
    ]
    u = hidden[0]
    for hop in range(T - 1):
        u = _tc_hop(e[hop], e[hop + 1], u)
    return u


# speedup vs baseline: 1.1160x; 1.1160x over previous
import functools

import jax
import jax.numpy as jnp
from jax import lax
from jax.experimental import pallas as pl
from jax.experimental.pallas import tpu as pltpu
from jax.experimental.pallas import tpu_sc as plsc

_D = 64
_S = 20
_NC = 2
_NS = 16
_NW = _NC * _NS

_K = 32
_ROWS = _K * _S
_IDXW = 128
_NG = _ROWS // _IDXW


def _sc_pool(story1d, c_flat, segs, toff):
    segs_per_w = segs // _NW
    chunks = segs_per_w // _K

    mesh = plsc.VectorSubcoreMesh(core_axis_name="c", subcore_axis_name="s")

    @functools.partial(
        pl.kernel,
        mesh=mesh,
        compiler_params=pltpu.CompilerParams(
            use_tc_tiling_on_sc=False, needs_layout_passes=False
        ),
        out_type=jax.ShapeDtypeStruct((segs, _D), jnp.bfloat16),
        scratch_types=[
            pltpu.VMEM((_ROWS,), jnp.int32),
            pltpu.VMEM((_ROWS,), jnp.int32),
            pltpu.VMEM((_ROWS, _D), jnp.bfloat16),
            pltpu.VMEM((_ROWS, _D), jnp.bfloat16),
            pltpu.VMEM((_K, _D), jnp.bfloat16),
            pltpu.VMEM((_K, _D), jnp.bfloat16),
            pltpu.SemaphoreType.DMA,
            pltpu.SemaphoreType.DMA,
            pltpu.SemaphoreType.DMA,
            pltpu.SemaphoreType.DMA,
            pltpu.SemaphoreType.DMA,
            pltpu.SemaphoreType.DMA,
        ],
    )
    def k(story_hbm, c_hbm, out_hbm,
          idx0, idx1, rows0, rows1, outv0, outv1,
          sem_i0, sem_i1, sem_r0, sem_r1, sem_o0, sem_o1):
        idx_b = [idx0, idx1]
        rows_b = [rows0, rows1]
        out_b = [outv0, outv1]
        sem_i = [sem_i0, sem_i1]
        sem_r = [sem_r0, sem_r1]
        sem_o = [sem_o0, sem_o1]

        wid = lax.axis_index("s") * _NC + lax.axis_index("c")
        seg0 = wid * segs_per_w

        def stage_idx(ci, b):
            pltpu.async_copy(
                story_hbm.at[pl.ds((seg0 + ci * _K) * _S, _ROWS)],
                idx_b[b], sem_i[b],
            )

        def fire_gathers(b):
            pltpu.make_async_copy(
                story_hbm.at[pl.ds(0, _ROWS)], idx_b[b], sem_i[b]
            ).wait()
            if toff:
                for i in range(_ROWS // 16):
                    sl = pl.ds(i * 16, 16)
                    idx_b[b][sl] = idx_b[b][sl] + toff
            for g in range(_NG):
                pltpu.async_copy(
                    c_hbm.at[idx_b[b].at[pl.ds(g * _IDXW, _IDXW)]],
                    rows_b[b].at[pl.ds(g * _IDXW, _IDXW)],
                    sem_r[b],
                )

        def drain_gathers(b):
            for g in range(_NG):
                pltpu.make_async_copy(
                    c_hbm.at[idx_b[b].at[pl.ds(g * _IDXW, _IDXW)]],
                    rows_b[b].at[pl.ds(g * _IDXW, _IDXW)],
                    sem_r[b],
                ).wait()

        def pool(b, ci):
            rows_v = rows_b[b]
            out_v = out_b[b]

            def seg_body(j, _):
                base = j * _S
                for l in range(_D // 32):
                    sl = pl.ds(l * 32, 32)
                    a0, a1 = plsc.unpack(
                        rows_v[base, sl], format=plsc.PackFormat.INTERLEAVED
                    )
                    for s in range(1, _S):
                        r0, r1 = plsc.unpack(
                            rows_v[base + s, sl],
                            format=plsc.PackFormat.INTERLEAVED,
                        )
                        a0 = a0 + r0
                        a1 = a1 + r1
                    out_v[j, sl] = plsc.pack(
                        a0, a1, format=plsc.PackFormat.INTERLEAVED
                    )
                return 0

            lax.fori_loop(0, _K, seg_body, 0, unroll=False)
            pltpu.async_copy(
                out_v,
                out_hbm.at[pl.ds(seg0 + ci * _K, _K)],
                sem_o[b],
            )

        def wait_out(b, ci):
            pltpu.make_async_copy(
                out_b[b],
                out_hbm.at[pl.ds(seg0 + ci * _K, _K)],
                sem_o[b],
            ).wait()

        stage_idx(0, 0)
        stage_idx(1, 1)
        fire_gathers(0)

        def chunk2_body(h, _):
            ci = h * 2
            for b in range(2):
                c = ci + b
                drain_gathers(b)

                @pl.when(c + 2 < chunks)
                def _():
                    stage_idx(c + 2, b)

                @pl.when(c + 1 < chunks)
                def _():
                    fire_gathers(1 - b)

                @pl.when(c >= 2)
                def _():
                    wait_out(b, c - 2)

                pool(b, c)
            return 0

        lax.fori_loop(0, chunks // 2, chunk2_body, 0, unroll=False)
        wait_out(0, chunks - 2)
        wait_out(1, chunks - 1)

    return k(story1d, c_flat)


def _tc_hop(ea, ec, u):
    B, M, D = ea.shape
    BB = 128

    def body(ea_ref, ec_ref, h_ref, o_ref):
        u = h_ref[...]
        eaf = ea_ref[...].astype(jnp.float32)
        logit = jnp.sum(eaf * u[:, None, :], axis=2)
        mx = jnp.max(logit, axis=1, keepdims=True)
        w = jnp.exp(logit - mx)
        den = jnp.sum(w, axis=1)
        ecf = ec_ref[...].astype(jnp.float32)
        num = jnp.sum(ecf * w[:, :, None], axis=1)
        o_ref[...] = u + num / den[:, None]

    return pl.pallas_call(
        body,
        grid=(B // BB,),
        in_specs=[
            pl.BlockSpec((BB, M, D), lambda i: (i, 0, 0)),
            pl.BlockSpec((BB, M, D), lambda i: (i, 0, 0)),
            pl.BlockSpec((BB, D), lambda i: (i, 0)),
        ],
        out_specs=pl.BlockSpec((BB, D), lambda i: (i, 0)),
        out_shape=jax.ShapeDtypeStruct((B, D), jnp.float32),
    )(ea, ec, u)


def kernel(story, hidden, C):
    B, M, S = story.shape
    T, vocab, D = C.shape
    story1d = story.reshape(-1).astype(jnp.int32)
    c_flat = C.astype(jnp.bfloat16).reshape(T * vocab, D)
    e = [
        _sc_pool(story1d, c_flat, B * M, t * vocab).reshape(B, M, D)
        for t in range(T)
    ]
    u = hidden[0]
    for hop in range(T - 1):
        u = _tc_hop(e[hop], e[hop + 1], u)
    return u
